# baseline (speedup 1.0000x reference)
import functools

import jax
import jax.numpy as jnp
from jax import lax
from jax.experimental import pallas as pl
from jax.experimental.pallas import tpu as pltpu
from jax.experimental.pallas import tpu_sc as plsc

B = 16384
C = 1000
D = 128

NC = 2
NS = 16
L = 16
NW = NC * NS

CP = 1024
WROWS = CP // NS
LASTROWS = C - (NS - 1) * WROWS
CHUNK = 128
NCHUNK = B // (NW * CHUNK)
NBUF = 2


@functools.partial(
    pl.kernel,
    mesh=plsc.VectorSubcoreMesh(core_axis_name="c", subcore_axis_name="s"),
    out_type=jax.ShapeDtypeStruct((B, D), jnp.float32),
    scratch_types=(
        [pltpu.VMEM((NCHUNK, CHUNK), jnp.int32),
         pltpu.VMEM((WROWS, D), jnp.float32),
         pltpu.VMEM_SHARED((CP, D), jnp.float32)]
        + [pltpu.VMEM((CHUNK, D), jnp.float32) for _ in range(3 * NBUF)]
        + [pltpu.SemaphoreType.DMA for _ in range(3 * NBUF)]
    ),
)
def _sc_weighter(feat_hbm, lab_hbm, w_hbm, out_hbm,
                 idx_v, wtile_v, sig_sh,
                 rows0, rows1, feat0, feat1, out0, out1,
                 sg0, sg1, sf0, sf1, so0, so1):
    rows_v = (rows0, rows1)
    feat_v = (feat0, feat1)
    out_v = (out0, out1)
    sem_g = (sg0, sg1)
    sem_f = (sf0, sf1)
    sem_o = (so0, so1)

    sid = lax.axis_index("s")
    wid = sid * NC + lax.axis_index("c")
    pltpu.sync_copy(lab_hbm.at[wid], idx_v)

    def row0_of(j):
        return (wid * NCHUNK + j) * CHUNK

    feats = {}
    outs = {}

    def issue_feat(j):
        p = j % NBUF
        feats[j] = pltpu.async_copy(feat_hbm.at[pl.ds(row0_of(j), CHUNK)],
                                    feat_v[p], sem_f[p])

    issue_feat(0)
    issue_feat(1)

    @pl.when(sid < NS - 1)
    def _():
        pltpu.sync_copy(w_hbm.at[pl.ds(sid * WROWS, WROWS)], wtile_v)

    @pl.when(sid == NS - 1)
    def _():
        pltpu.sync_copy(w_hbm.at[pl.ds((NS - 1) * WROWS, LASTROWS)],
                        wtile_v.at[pl.ds(0, LASTROWS)])

    def sig_body(r, carry):
        for c in range(D // L):
            w = wtile_v[r, pl.ds(c * L, L)]
            wtile_v[r, pl.ds(c * L, L)] = 1.0 / (1.0 + jnp.exp(-w))
        return carry

    lax.fori_loop(0, WROWS, sig_body, 0)

    @pl.when(sid < NS - 1)
    def _():
        pltpu.sync_copy(wtile_v, sig_sh.at[pl.ds(sid * WROWS, WROWS)])

    @pl.when(sid == NS - 1)
    def _():
        pltpu.sync_copy(wtile_v.at[pl.ds(0, LASTROWS)],
                        sig_sh.at[pl.ds((NS - 1) * WROWS, LASTROWS)])

    plsc.subcore_barrier()

    gathers = {}

    def issue_gather(j):
        p = j % NBUF
        gathers[j] = pltpu.async_copy(sig_sh.at[idx_v.at[j]], rows_v[p], sem_g[p])

    issue_gather(0)
    issue_gather(1)

    for j in range(NCHUNK):
        p = j % NBUF
        gathers[j].wait()
        feats[j].wait()
        if j >= NBUF:
            outs[j - NBUF].wait()

        def row_body(r, carry):
            for c in range(D // L):
                s = rows_v[p][r, pl.ds(c * L, L)]
                f = feat_v[p][r, pl.ds(c * L, L)]
                out_v[p][r, pl.ds(c * L, L)] = f * s
            return carry

        lax.fori_loop(0, CHUNK, row_body, 0)
        outs[j] = pltpu.async_copy(out_v[p], out_hbm.at[pl.ds(row0_of(j), CHUNK)],
                                   sem_o[p])
        if j + NBUF < NCHUNK:
            issue_feat(j + NBUF)
            issue_gather(j + NBUF)

    for j in range(NCHUNK - NBUF, NCHUNK):
        outs[j].wait()


def kernel(features, labels, W):
    lab = labels.astype(jnp.int32).reshape(NW, NCHUNK, CHUNK)
    return _sc_weighter(features, lab, W)


# speedup vs baseline: 1.0285x; 1.0285x over previous
import functools

import jax
import jax.numpy as jnp
from jax import lax
from jax.experimental import pallas as pl
from jax.experimental.pallas import tpu as pltpu
from jax.experimental.pallas import tpu_sc as plsc

B = 16384
C = 1000
D = 128

NC = 2
NS = 16
L = 16
NW = NC * NS

CP = 1024
WROWS = CP // NS
LASTROWS = C - (NS - 1) * WROWS
CHUNK = 128
NCHUNK = B // (NW * CHUNK)
NBUF = 2

_HI = -65536
_bc = lax.bitcast_convert_type


@functools.partial(
    pl.kernel,
    mesh=plsc.VectorSubcoreMesh(core_axis_name="c", subcore_axis_name="s"),
    out_type=jax.ShapeDtypeStruct((B, D), jnp.float32),
    scratch_types=(
        [pltpu.VMEM((NCHUNK, CHUNK), jnp.int32),
         pltpu.VMEM((WROWS, D), jnp.float32),
         pltpu.VMEM((WROWS, D), jnp.int32),
         pltpu.VMEM_SHARED((CP, D), jnp.int32),
         pltpu.VMEM((CHUNK, D), jnp.int32),
         pltpu.VMEM((CHUNK, D), jnp.int32)]
        + [pltpu.VMEM((CHUNK, D), jnp.float32) for _ in range(2 * NBUF)]
        + [pltpu.SemaphoreType.DMA for _ in range(3 * NBUF)]
    ),
)
def _sc_weighter(feat_hbm, lab_hbm, w_hbm, out_hbm,
                 idx_v, wtile_f, wtile_p, sig_sh,
                 rows0, rows1, feat0, feat1, out0, out1,
                 sg0, sg1, sf0, sf1, so0, so1):
    rows_v = (rows0, rows1)
    feat_v = (feat0, feat1)
    out_v = (out0, out1)
    sem_g = (sg0, sg1)
    sem_f = (sf0, sf1)
    sem_o = (so0, so1)

    sid = lax.axis_index("s")
    wid = sid * NC + lax.axis_index("c")
    pltpu.sync_copy(lab_hbm.at[wid], idx_v)

    def row0_of(j):
        return (wid * NCHUNK + j) * CHUNK

    feats = {}
    outs = {}

    def issue_feat(j):
        p = j % NBUF
        feats[j] = pltpu.async_copy(feat_hbm.at[pl.ds(row0_of(j), CHUNK)],
                                    feat_v[p], sem_f[p])

    issue_feat(0)
    issue_feat(1)

    @pl.when(sid < NS - 1)
    def _():
        pltpu.sync_copy(w_hbm.at[pl.ds(sid * WROWS, WROWS)], wtile_f)

    @pl.when(sid == NS - 1)
    def _():
        pltpu.sync_copy(w_hbm.at[pl.ds((NS - 1) * WROWS, LASTROWS)],
                        wtile_f.at[pl.ds(0, LASTROWS)])

    def sig_body(r, carry):
        for c in range(D // (2 * L)):
            s0 = 1.0 / (1.0 + jnp.exp(-wtile_f[r, pl.ds(2 * c * L, L)]))
            s1 = 1.0 / (1.0 + jnp.exp(-wtile_f[r, pl.ds((2 * c + 1) * L, L)]))
            lo = lax.shift_right_logical(_bc(s0, jnp.int32) + 0x8000, 16)
            hi = jnp.bitwise_and(_bc(s1, jnp.int32) + 0x8000, _HI)
            wtile_p[r, pl.ds(c * L, L)] = jnp.bitwise_or(lo, hi)
        return carry

    lax.fori_loop(0, WROWS, sig_body, 0)

    @pl.when(sid < NS - 1)
    def _():
        pltpu.sync_copy(wtile_p, sig_sh.at[pl.ds(sid * WROWS, WROWS)])

    @pl.when(sid == NS - 1)
    def _():
        pltpu.sync_copy(wtile_p.at[pl.ds(0, LASTROWS)],
                        sig_sh.at[pl.ds((NS - 1) * WROWS, LASTROWS)])

    plsc.subcore_barrier()

    gathers = {}

    def issue_gather(j):
        p = j % NBUF
        gathers[j] = pltpu.async_copy(sig_sh.at[idx_v.at[j]], rows_v[p], sem_g[p])

    issue_gather(0)
    issue_gather(1)

    for j in range(NCHUNK):
        p = j % NBUF
        gathers[j].wait()
        feats[j].wait()
        if j >= NBUF:
            outs[j - NBUF].wait()

        def row_body(r, carry):
            for c in range(D // (2 * L)):
                w = rows_v[p][r, pl.ds(c * L, L)]
                s0 = _bc(lax.shift_left(w, 16), jnp.float32)
                s1 = _bc(jnp.bitwise_and(w, _HI), jnp.float32)
                f0 = feat_v[p][r, pl.ds(2 * c * L, L)]
                f1 = feat_v[p][r, pl.ds((2 * c + 1) * L, L)]
                out_v[p][r, pl.ds(2 * c * L, L)] = f0 * s0
                out_v[p][r, pl.ds((2 * c + 1) * L, L)] = f1 * s1
            return carry

        lax.fori_loop(0, CHUNK, row_body, 0)
        outs[j] = pltpu.async_copy(out_v[p], out_hbm.at[pl.ds(row0_of(j), CHUNK)],
                                   sem_o[p])
        if j + NBUF < NCHUNK:
            issue_feat(j + NBUF)
            issue_gather(j + NBUF)

    for j in range(NCHUNK - NBUF, NCHUNK):
        outs[j].wait()


def kernel(features, labels, W):
    lab = labels.astype(jnp.int32).reshape(NW, NCHUNK, CHUNK)
    return _sc_weighter(features, lab, W)
